# baseline (speedup 1.0000x reference)
import functools

import jax
import jax.numpy as jnp
from jax import lax
from jax.experimental import pallas as pl
from jax.experimental.pallas import tpu as pltpu
from jax.experimental.pallas import tpu_sc as plsc

N = 10000
D = 128
E = 160000
H = 16

NC = 2
NS = 16
NW = NC * NS
EPT = 5120
EPAD = EPT * NW
CHUNK = 128
NCH = EPT // CHUNK

NEXT = N + 16
ZROWS = 626
ACC_ROWS = NS * ZROWS
OUT_ROWS = N // NS
MROWS = 640


@functools.cache
def _sc_mesh():
    return plsc.VectorSubcoreMesh(core_axis_name="c", subcore_axis_name="s",
                                  num_cores=NC, num_subcores=NS)


DH = D // NC
WCH = DH + 16
TCH = EPAD // NS // CHUNK
NBUF = 4


@functools.cache
def _build_sc_aggregate():
    @functools.partial(
        pl.kernel,
        out_type=jax.ShapeDtypeStruct((NC, N, WCH), jnp.float32),
        mesh=_sc_mesh(),
        compiler_params=pltpu.CompilerParams(
            use_tc_tiling_on_sc=False, needs_layout_passes=False),
        scratch_types=[
            pltpu.VMEM((TCH, CHUNK), jnp.int32),
            pltpu.VMEM((TCH, CHUNK), jnp.int32),
            [pltpu.VMEM((CHUNK, WCH), jnp.float32) for _ in range(NBUF)],
            pltpu.VMEM((CHUNK, WCH), jnp.float32),
            pltpu.VMEM_SHARED((ACC_ROWS, WCH), jnp.float32),
            pltpu.SemaphoreType.DMA,
            pltpu.SemaphoreType.DMA,
        ],
    )
    def sc_aggregate(x2_hbm, src_hbm, dst_hbm, z_hbm,
                     agg_out, sidx, didx, rows, zb, acc_sh, gsem, ssem):
        c = lax.axis_index("c")
        s = lax.axis_index("s")

        pltpu.sync_copy(z_hbm, zb)
        for k in range(4):
            pltpu.sync_copy(zb, acc_sh.at[pl.ds(s * ZROWS + k * CHUNK,
                                                CHUNK)])
        pltpu.sync_copy(zb.at[pl.ds(0, ZROWS - 4 * CHUNK)],
                        acc_sh.at[pl.ds(s * ZROWS + 4 * CHUNK,
                                        ZROWS - 4 * CHUNK)])
        pltpu.sync_copy(src_hbm.at[pl.ds(s * TCH, TCH)], sidx)
        pltpu.sync_copy(dst_hbm.at[pl.ds(s * TCH, TCH)], didx)
        plsc.subcore_barrier()

        def gather(j, b):
            pltpu.async_copy(x2_hbm.at[c].at[sidx.at[j]], rows[b], gsem)

        def scatter(j, b):
            pltpu.async_copy(rows[b], acc_sh.at[didx.at[j]], ssem, add=True)

        def wait_gather(b):
            pltpu.make_async_copy(z_hbm, rows[b], gsem).wait()

        def wait_scatter(b):
            pltpu.make_async_copy(z_hbm, rows[b], ssem).wait()

        for b in range(NBUF):
            gather(b, b)

        def group(g, carry):
            base = g * NBUF
            for b in range(NBUF):
                wait_gather(b)
                scatter(base + b, b)
            for b in range(NBUF):
                wait_scatter(b)

                @pl.when(base + b + NBUF < TCH)
                def _(b=b):
                    gather(base + b + NBUF, b)

            return carry

        lax.fori_loop(0, TCH // NBUF, group, 0)
        plsc.subcore_barrier()

        pltpu.sync_copy(acc_sh.at[pl.ds(s * OUT_ROWS, OUT_ROWS)],
                        agg_out.at[c, pl.ds(s * OUT_ROWS, OUT_ROWS), :])

    return sc_aggregate


_RB = 400
_NB = N // _RB


def _tc_xw_body(x_ref, w_ref, y_ref):
    y_ref[...] = jnp.dot(x_ref[...], w_ref[...],
                         preferred_element_type=jnp.float32)


def _tc_xw(x, w):
    return pl.pallas_call(
        _tc_xw_body,
        grid=(_NB,),
        in_specs=[pl.BlockSpec((_RB, D), lambda i: (i, 0)),
                  pl.BlockSpec((D, D), lambda i: (0, 0))],
        out_specs=pl.BlockSpec((_RB, D), lambda i: (i, 0)),
        out_shape=jax.ShapeDtypeStruct((N, D), jnp.float32),
    )(x, w)


def _tc_dense_body(y_ref, aggp_ref, w_ref, b_ref, ew1h_ref,
                   tw1q_ref, ew1g_ref, tw1a_ref, eb1_ref,
                   emb_ref, p_ref, q_ref, c1_ref, t1g_ref, gacc):
    i = pl.program_id(0)
    aggp = aggp_ref[...]
    agg = jnp.concatenate([aggp[0, :, :DH], aggp[1, :, :DH]], axis=1)
    deg = aggp[0, :, DH:DH + 1]
    pre = agg / jnp.maximum(deg, 1.0)
    emb = jnp.tanh(
        y_ref[...]
        + jnp.dot(pre, w_ref[...], preferred_element_type=jnp.float32)
        + b_ref[...])
    emb_ref[...] = emb
    p_ref[...] = jnp.dot(emb, ew1h_ref[...],
                         preferred_element_type=jnp.float32)
    q_ref[...] = jnp.dot(emb, tw1q_ref[...],
                         preferred_element_type=jnp.float32)

    @pl.when(i == 0)
    def _():
        gacc[...] = jnp.zeros_like(gacc)

    gacc[...] += jnp.sum(emb, axis=0, keepdims=True)

    @pl.when(i == _NB - 1)
    def _():
        g = gacc[...] / jnp.float32(N)
        c1_ref[...] = jnp.dot(
            g, ew1g_ref[...], preferred_element_type=jnp.float32) + eb1_ref[...]
        t1g_ref[...] = jnp.dot(
            g, tw1a_ref[...], preferred_element_type=jnp.float32)


def _tc_dense(y, aggp, w, b, ew1h, tw1q, ew1g, tw1a, eb1):
    return pl.pallas_call(
        _tc_dense_body,
        grid=(_NB,),
        in_specs=[
            pl.BlockSpec((_RB, D), lambda i: (i, 0)),
            pl.BlockSpec((NC, _RB, WCH), lambda i: (0, i, 0)),
            pl.BlockSpec((D, D), lambda i: (0, 0)),
            pl.BlockSpec((1, D), lambda i: (0, 0)),
            pl.BlockSpec((D, H), lambda i: (0, 0)),
            pl.BlockSpec((D, H), lambda i: (0, 0)),
            pl.BlockSpec((D, H), lambda i: (0, 0)),
            pl.BlockSpec((D, H), lambda i: (0, 0)),
            pl.BlockSpec((1, H), lambda i: (0, 0)),
        ],
        out_specs=[
            pl.BlockSpec((_RB, D), lambda i: (i, 0)),
            pl.BlockSpec((_RB, H), lambda i: (i, 0)),
            pl.BlockSpec((_RB, H), lambda i: (i, 0)),
            pl.BlockSpec((1, H), lambda i: (0, 0)),
            pl.BlockSpec((1, H), lambda i: (0, 0)),
        ],
        out_shape=[
            jax.ShapeDtypeStruct((N, D), jnp.float32),
            jax.ShapeDtypeStruct((N, H), jnp.float32),
            jax.ShapeDtypeStruct((N, H), jnp.float32),
            jax.ShapeDtypeStruct((1, H), jnp.float32),
            jax.ShapeDtypeStruct((1, H), jnp.float32),
        ],
        scratch_shapes=[pltpu.VMEM((1, D), jnp.float32)],
    )(y, aggp, w, b, ew1h, tw1q, ew1g, tw1a, eb1)


@functools.cache
def _build_sc_pairs():
    @functools.partial(
        pl.kernel,
        out_type=(
            jax.ShapeDtypeStruct((EPAD, H), jnp.float32),
            jax.ShapeDtypeStruct((EPAD, H), jnp.float32),
        ),
        mesh=_sc_mesh(),
        compiler_params=pltpu.CompilerParams(
            use_tc_tiling_on_sc=False, needs_layout_passes=False),
        scratch_types=[
            pltpu.VMEM((NCH, CHUNK), jnp.int32),
            pltpu.VMEM((NCH, CHUNK), jnp.int32),
            pltpu.VMEM((EPT, H), jnp.float32),
            pltpu.VMEM_SHARED((NEXT, H), jnp.float32),
            pltpu.SemaphoreType.DMA,
        ],
    )
    def sc_pairs(p_hbm, src_hbm, dst_hbm, zs_out, zd_out, sidx, didx, zbuf,
                 p_sh, sem):
        c = lax.axis_index("c")
        s = lax.axis_index("s")
        wid = s * NC + c
        prt = NEXT // NS
        pltpu.sync_copy(p_hbm.at[pl.ds(s * prt, prt)],
                        p_sh.at[pl.ds(s * prt, prt)])
        pltpu.sync_copy(src_hbm.at[pl.ds(wid * NCH, NCH)], sidx)
        pltpu.sync_copy(dst_hbm.at[pl.ds(wid * NCH, NCH)], didx)
        plsc.subcore_barrier()

        def gather(idx, out_hbm):
            def fire(j, carry):
                pltpu.async_copy(p_sh.at[idx.at[j]],
                                 zbuf.at[pl.ds(j * CHUNK, CHUNK)], sem)
                return carry
            lax.fori_loop(0, NCH, fire, 0)

            def drain(j, carry):
                pltpu.make_async_copy(p_hbm.at[pl.ds(0, CHUNK)],
                                      zbuf.at[pl.ds(0, CHUNK)], sem).wait()
                return carry
            lax.fori_loop(0, NCH, drain, 0)
            pltpu.sync_copy(zbuf, out_hbm.at[pl.ds(wid * EPT, EPT)])

        gather(sidx, zs_out)
        gather(didx, zd_out)

    return sc_pairs


_ERB = 2048
_ENB = (EPAD // 8) // _ERB
_EROWS = E // 8


def _tc_edge_body(zs_ref, zd_ref, g1_ref, c1t_ref, w2t_ref, g8_ref,
                  srcr_ref, dstr_ref, emb_ref, tw1b_ref, tw1c_ref,
                  tb1_ref, t1g_ref,
                  vfs_ref, lpe_ref, c2_ref, smf, smi):
    i = pl.program_id(0)

    @pl.when(i == 0)
    def _():
        smf[0] = jnp.float32(-1e30)
        smf[1] = jnp.float32(0.0)
        smf[2] = jnp.float32(-3e38)
        smf[3] = jnp.float32(0.0)
        smi[0] = jnp.int32(0)

    z = zs_ref[...] + zd_ref[...] + c1t_ref[...]
    sig = 1.0 / (1.0 + jnp.exp(-z))
    scores = jnp.dot(sig * w2t_ref[...], g8_ref[...],
                     preferred_element_type=jnp.float32)

    rid = lax.broadcasted_iota(jnp.int32, (_ERB, 8), 0) + i * _ERB
    valid = rid < _EROWS
    sc = jnp.where(valid, scores, -1e30)

    bm = jnp.max(sc)
    m0 = smf[0]
    mn = jnp.maximum(m0, bm)
    seb = jnp.sum(jnp.exp(sc - mn))
    smf[1] = smf[1] * jnp.exp(m0 - mn) + seb
    smf[0] = mn

    eid = rid * 8 + lax.broadcasted_iota(jnp.int32, (_ERB, 8), 1)
    vv = jnp.where(valid, g1_ref[...] + scores, -3e38)
    bv = jnp.max(vv)
    bid = jnp.min(jnp.where(vv == bv, eid, jnp.int32(2**30)))
    bsc = jnp.max(jnp.where((vv == bv) & (eid == bid), scores, -3e38))

    @pl.when(bv > smf[2])
    def _():
        smf[2] = bv
        smf[3] = bsc
        smi[0] = bid

    @pl.when(i == _ENB - 1)
    def _():
        lse = smf[0] + jnp.log(smf[1])
        lpe_ref[...] = jnp.broadcast_to(smf[3] - lse, (1, 1))
        e_idx = smi[0]
        row = e_idx // 128
        col = e_idx % 128
        lanes = lax.broadcasted_iota(jnp.int32, (1, 128), 1)
        drow = dstr_ref[pl.ds(row, 1), :]
        srow = srcr_ref[pl.ds(row, 1), :]
        v_fir = jnp.sum(jnp.where(lanes == col, drow, 0))
        v_sec = jnp.sum(jnp.where(lanes == col, srow, 0))
        lanes2 = lax.broadcasted_iota(jnp.int32, (1, 2), 1)
        vfs_ref[...] = jnp.where(lanes2 == 0, v_fir, v_sec)
        evf = emb_ref[pl.ds(v_fir, 1), :]
        evs = emb_ref[pl.ds(v_sec, 1), :]
        c2_ref[...] = (
            t1g_ref[...]
            + jnp.dot(evf + evs, tw1b_ref[...],
                      preferred_element_type=jnp.float32)
            + jnp.dot(evf, tw1c_ref[...],
                      preferred_element_type=jnp.float32)
            + tb1_ref[...])


def _tc_edge(zs, zd, g1p, c1t, w2t, g8, srcr, dstr, emb, tw1b, tw1c, tb1,
             t1g):
    def full(shape):
        return pl.BlockSpec(shape, lambda i: tuple(0 for _ in shape))
    return pl.pallas_call(
        _tc_edge_body,
        grid=(_ENB,),
        in_specs=[
            pl.BlockSpec((_ERB, D), lambda i: (i, 0)),
            pl.BlockSpec((_ERB, D), lambda i: (i, 0)),
            pl.BlockSpec((_ERB, 8), lambda i: (i, 0)),
            full((1, D)),
            full((1, D)),
            full((D, 8)),
            full((E // 128, 128)),
            full((E // 128, 128)),
            full((N, D)),
            full((D, H)),
            full((D, H)),
            full((1, H)),
            full((1, H)),
        ],
        out_specs=[
            full((1, 2)),
            full((1, 1)),
            full((1, H)),
        ],
        out_shape=[
            jax.ShapeDtypeStruct((1, 2), jnp.int32),
            jax.ShapeDtypeStruct((1, 1), jnp.float32),
            jax.ShapeDtypeStruct((1, H), jnp.float32),
        ],
        scratch_shapes=[
            pltpu.SMEM((4,), jnp.float32),
            pltpu.SMEM((2,), jnp.int32),
        ],
    )(zs, zd, g1p, c1t, w2t, g8, srcr, dstr, emb, tw1b, tw1c, tb1, t1g)


@functools.cache
def _build_sc_mask():
    @functools.partial(
        pl.kernel,
        out_type=jax.ShapeDtypeStruct((NC, MROWS, 16), jnp.float32),
        mesh=_sc_mesh(),
        compiler_params=pltpu.CompilerParams(
            use_tc_tiling_on_sc=False, needs_layout_passes=False),
        scratch_types=[
            pltpu.VMEM((NCH, CHUNK), jnp.int32),
            pltpu.VMEM((NCH, CHUNK), jnp.int32),
            pltpu.VMEM((16,), jnp.int32),
            pltpu.VMEM((MROWS // CHUNK, CHUNK), jnp.int32),
            pltpu.VMEM((MROWS, 16), jnp.float32),
            pltpu.VMEM_SHARED((MROWS, 16), jnp.float32),
        ],
    )
    def sc_mask(src_hbm, dst_hbm, vf_hbm, idx_hbm, mask_out,
                sbuf, dbuf, vfb, idxb, mbuf, msh):
        c = lax.axis_index("c")
        s = lax.axis_index("s")
        wid = s * NC + c
        rows_per_tile = MROWS // NS

        def zbody(i, carry):
            mbuf[i, :] = jnp.zeros((16,), jnp.float32)
            return carry
        lax.fori_loop(0, MROWS, zbody, 0)

        pltpu.sync_copy(mbuf.at[pl.ds(s * rows_per_tile, rows_per_tile)],
                        msh.at[pl.ds(s * rows_per_tile, rows_per_tile)])
        pltpu.sync_copy(src_hbm.at[pl.ds(wid * NCH, NCH)], sbuf)
        pltpu.sync_copy(dst_hbm.at[pl.ds(wid * NCH, NCH)], dbuf)
        pltpu.sync_copy(vf_hbm, vfb)
        pltpu.sync_copy(idx_hbm, idxb)
        plsc.subcore_barrier()

        vfv = vfb[...]
        ones16 = jnp.ones((16,), jnp.float32)

        def body(t, carry):
            j = t // 8
            k = t % 8
            sv = sbuf[j, pl.ds(k * 16, 16)]
            dv = dbuf[j, pl.ds(k * 16, 16)]
            hit = sv == vfv
            plsc.store_scatter(mbuf, [dv // 16, dv % 16], ones16, mask=hit)
            return carry
        lax.fori_loop(0, NCH * 8, body, 0)
        plsc.subcore_barrier()

        def abody(k, carry):
            pltpu.sync_copy(mbuf.at[pl.ds(k * CHUNK, CHUNK)],
                            msh.at[idxb.at[k]], add=True)
            return carry
        lax.fori_loop(0, MROWS // CHUNK, abody, 0)
        plsc.subcore_barrier()

        pltpu.sync_copy(
            msh.at[pl.ds(s * rows_per_tile, rows_per_tile)],
            mask_out.at[c, pl.ds(s * rows_per_tile, rows_per_tile), :])

    return sc_mask


def _tc_third_body(q_ref, mask_ref, g2_ref, c2t_ref, w3t_ref, g8_ref,
                   vf_ref, lpe_ref, vthi_ref, lp_ref):
    z = q_ref[...] + c2t_ref[...]
    sig = 1.0 / (1.0 + jnp.exp(-z))
    scores = jnp.dot(sig * w3t_ref[...], g8_ref[...],
                     preferred_element_type=jnp.float32)
    hits = mask_ref[0] + mask_ref[1]
    nid = (lax.broadcasted_iota(jnp.int32, (N // 8, 8), 0) * 8
           + lax.broadcasted_iota(jnp.int32, (N // 8, 8), 1))
    vf = vf_ref[0, 0]
    masked = jnp.where((hits > 0.5) | (nid == vf), jnp.float32(-1e9), scores)
    m3 = jnp.max(masked)
    sh = masked - m3
    logp = sh - jnp.log(jnp.sum(jnp.exp(sh)))
    v = g2_ref[...] + logp
    bv = jnp.max(v)
    vthi = jnp.min(jnp.where(v == bv, nid, jnp.int32(2**30)))
    lp3 = jnp.max(jnp.where((v == bv) & (nid == vthi), logp,
                            jnp.float32(-3e38)))
    vthi_ref[...] = jnp.broadcast_to(vthi, (1, 1))
    lp_ref[...] = jnp.broadcast_to(lpe_ref[0, 0] + lp3, (1, 1))


def _tc_third(qr, mask2, g2r, c2t, w3t, g8, vfr, lpe):
    def full(shape):
        return pl.BlockSpec(shape, lambda: tuple(0 for _ in shape))
    return pl.pallas_call(
        _tc_third_body,
        in_specs=[
            full((N // 8, 128)),
            full((NC, N // 8, 8)),
            full((N // 8, 8)),
            full((1, 128)),
            full((1, 128)),
            full((128, 8)),
            full((1, 1)),
            full((1, 1)),
        ],
        out_specs=[full((1, 1)), full((1, 1))],
        out_shape=[
            jax.ShapeDtypeStruct((1, 1), jnp.int32),
            jax.ShapeDtypeStruct((1, 1), jnp.float32),
        ],
    )(qr, mask2, g2r, c2t, w3t, g8, vfr, lpe)


def kernel(x, edge_index, W_gnn, b_gnn, eW1, eb1, eW2, eb2, tW1, tb1, tW2,
           tb2):
    f32 = jnp.float32
    src = edge_index[0]
    dst = edge_index[1]
    pad = EPAD - E
    padv = jnp.full((pad,), N, jnp.int32)
    srcp = jnp.concatenate([src, padv]).reshape(EPAD // CHUNK, CHUNK)
    dstp = jnp.concatenate([dst, padv]).reshape(EPAD // CHUNK, CHUNK)
    ones_col = jnp.ones((N, 16), f32)
    x2 = jnp.stack([jnp.concatenate([x[:, :DH], ones_col], axis=1),
                    jnp.concatenate([x[:, DH:], ones_col], axis=1)])
    x2 = jnp.concatenate([x2, jnp.zeros((NC, NEXT - N, WCH), f32)], axis=1)

    zrows = jnp.zeros((CHUNK, WCH), f32)
    y_xw = _tc_xw(x, W_gnn)
    aggp = _build_sc_aggregate()(x2, srcp, dstp, zrows)

    emb, P, Q, c1, t1g = _tc_dense(
        y_xw, aggp, W_gnn, b_gnn.reshape(1, D),
        eW1[D:2 * D], tW1[3 * D:4 * D], eW1[:D], tW1[:D],
        eb1.reshape(1, H))

    P_ext = jnp.concatenate([P, jnp.zeros((NEXT - N, H), f32)])
    Zs, Zd = _build_sc_pairs()(P_ext, srcp, dstp)

    g1 = jax.random.gumbel(jax.random.key(42), (E,), f32)
    g1p = jnp.concatenate([g1, jnp.full((pad,), -1e30, f32)])
    g1p = g1p.reshape(EPAD // 8, 8)
    g8 = (lax.broadcasted_iota(jnp.int32, (D, 8), 0) // H
          == lax.broadcasted_iota(jnp.int32, (D, 8), 1)).astype(f32)
    c1t = jnp.tile(c1, (1, 8))
    w2t = jnp.tile(eW2[:, 0].reshape(1, H), (1, 8))

    vfs, lpe, c2 = _tc_edge(
        Zs.reshape(EPAD // 8, D), Zd.reshape(EPAD // 8, D), g1p, c1t, w2t,
        g8, src.reshape(E // 128, 128), dst.reshape(E // 128, 128), emb,
        tW1[D:2 * D], tW1[2 * D:3 * D], tb1.reshape(1, H), t1g)

    vf16 = jnp.broadcast_to(vfs[0, 0], (16,)).astype(jnp.int32)
    idx640 = jnp.arange(MROWS, dtype=jnp.int32).reshape(MROWS // CHUNK, CHUNK)
    maskp = _build_sc_mask()(srcp, dstp, vf16, idx640)
    mask2 = maskp.reshape(NC, MROWS * 16)[:, :N].reshape(NC, N // 8, 8)

    g2 = jax.random.gumbel(jax.random.key(43), (N,), f32).reshape(N // 8, 8)
    c2t = jnp.tile(c2, (1, 8))
    w3t = jnp.tile(tW2[:, 0].reshape(1, H), (1, 8))

    vthi, lp = _tc_third(Q.reshape(N // 8, 128), mask2, g2, c2t, w3t, g8,
                         vfs[:, :1], lpe)

    action = jnp.stack([vfs[0, 0], vfs[0, 1], vthi[0, 0]])
    return action, lp[0, 0]


# speedup vs baseline: 1.1167x; 1.1167x over previous
import functools

import jax
import jax.numpy as jnp
from jax import lax
from jax.experimental import pallas as pl
from jax.experimental.pallas import tpu as pltpu
from jax.experimental.pallas import tpu_sc as plsc

N = 10000
D = 128
E = 160000
H = 16

NC = 2
NS = 16
NW = NC * NS
EPT = 5120
EPAD = EPT * NW
CHUNK = 128
NCH = EPT // CHUNK

NEXT = N + 16
ZROWS = 626
ACC_ROWS = NS * ZROWS
OUT_ROWS = N // NS
MROWS = 640


@functools.cache
def _sc_mesh():
    return plsc.VectorSubcoreMesh(core_axis_name="c", subcore_axis_name="s",
                                  num_cores=NC, num_subcores=NS)


DH = D // NC
WCH = DH + 16
TCH = EPAD // NS // CHUNK
NBUF = 4


WQ = D // NC // 2
NEXT2 = 10240
TSTG = NEXT2 // CHUNK // NS


@functools.cache
def _build_sc_aggregate():
    @functools.partial(
        pl.kernel,
        out_type=(
            jax.ShapeDtypeStruct((NC, N, WQ), jnp.float32),
            jax.ShapeDtypeStruct((N, 16), jnp.float32),
        ),
        mesh=_sc_mesh(),
        compiler_params=pltpu.CompilerParams(
            use_tc_tiling_on_sc=False, needs_layout_passes=False),
        scratch_types=[
            pltpu.VMEM((TCH, CHUNK), jnp.int32),
            pltpu.VMEM((TCH, CHUNK), jnp.int32),
            [pltpu.VMEM((CHUNK, WQ), jnp.float32) for _ in range(NBUF)],
            pltpu.VMEM((TSTG, CHUNK), jnp.int32),
            pltpu.VMEM((CHUNK, WQ), jnp.float32),
            pltpu.VMEM((CHUNK, WQ), jnp.float32),
            pltpu.VMEM((CHUNK, 16), jnp.float32),
            pltpu.VMEM((CHUNK, 16), jnp.float32),
            pltpu.VMEM_SHARED((NEXT2, WQ), jnp.float32),
            pltpu.VMEM_SHARED((ACC_ROWS, WQ), jnp.float32),
            pltpu.VMEM_SHARED((ACC_ROWS, 16), jnp.float32),
            pltpu.SemaphoreType.DMA,
            pltpu.SemaphoreType.DMA,
            pltpu.SemaphoreType.DMA,
        ],
    )
    def sc_aggregate(x4_hbm, src_hbm, dst_hbm, ids_hbm, z_hbm, z16_hbm,
                     agg_out, cnt_out, sidx, didx, rows, idb, tbuf, zb, zb16,
                     ones_v, tb_sh, acc_sh, cnt_sh, gsem, ssem, csem):
        c = lax.axis_index("c")
        s = lax.axis_index("s")

        pltpu.sync_copy(ids_hbm.at[pl.ds(s * TSTG, TSTG)], idb)

        def stage(k, carry):
            j = s * TSTG + k
            pltpu.async_copy(x4_hbm.at[c].at[idb.at[k]], tbuf, gsem).wait()
            pltpu.sync_copy(tbuf, tb_sh.at[pl.ds(j * CHUNK, CHUNK)])
            return carry
        lax.fori_loop(0, TSTG, stage, 0)

        def fill_ones(i, carry):
            ones_v[i, :] = jnp.ones((16,), jnp.float32)
            return carry
        lax.fori_loop(0, CHUNK, fill_ones, 0)

        pltpu.sync_copy(z_hbm, zb)
        pltpu.sync_copy(z16_hbm, zb16)
        for k in range(4):
            pltpu.sync_copy(zb, acc_sh.at[pl.ds(s * ZROWS + k * CHUNK,
                                                CHUNK)])
            pltpu.sync_copy(zb16, cnt_sh.at[pl.ds(s * ZROWS + k * CHUNK,
                                                  CHUNK)])
        tail = ZROWS - 4 * CHUNK
        pltpu.sync_copy(zb.at[pl.ds(0, tail)],
                        acc_sh.at[pl.ds(s * ZROWS + 4 * CHUNK, tail)])
        pltpu.sync_copy(zb16.at[pl.ds(0, tail)],
                        cnt_sh.at[pl.ds(s * ZROWS + 4 * CHUNK, tail)])
        pltpu.sync_copy(src_hbm.at[pl.ds(s * TCH, TCH)], sidx)
        pltpu.sync_copy(dst_hbm.at[pl.ds(s * TCH, TCH)], didx)
        plsc.subcore_barrier()

        def gather(j, b):
            pltpu.async_copy(tb_sh.at[sidx.at[j]], rows[b], gsem)

        def scatter(j, b):
            pltpu.async_copy(rows[b], acc_sh.at[didx.at[j]], ssem, add=True)
            pltpu.async_copy(ones_v, cnt_sh.at[didx.at[j]], csem, add=True)

        def wait_gather(b):
            pltpu.make_async_copy(z_hbm, rows[b], gsem).wait()

        def wait_scatter(b):
            pltpu.make_async_copy(z_hbm, rows[b], ssem).wait()

        for b in range(NBUF):
            gather(b, b)

        def group(g, carry):
            base = g * NBUF
            for b in range(NBUF):
                wait_gather(b)
                scatter(base + b, b)
            for b in range(NBUF):
                wait_scatter(b)

                @pl.when(base + b + NBUF < TCH)
                def _(b=b):
                    gather(base + b + NBUF, b)

            return carry

        lax.fori_loop(0, TCH // NBUF, group, 0)

        def cdrain(j, carry):
            pltpu.make_async_copy(z16_hbm, ones_v, csem).wait()
            return carry
        lax.fori_loop(0, TCH, cdrain, 0)
        plsc.subcore_barrier()

        pltpu.sync_copy(acc_sh.at[pl.ds(s * OUT_ROWS, OUT_ROWS)],
                        agg_out.at[c, pl.ds(s * OUT_ROWS, OUT_ROWS), :])

        @pl.when(c == 0)
        def _():
            pltpu.sync_copy(cnt_sh.at[pl.ds(s * OUT_ROWS, OUT_ROWS)],
                            cnt_out.at[pl.ds(s * OUT_ROWS, OUT_ROWS), :])

    return sc_aggregate


_RB = 400
_NB = N // _RB


def _tc_dense_body(x_ref, agga_ref, aggb_ref, cnt_ref, w_ref, b_ref,
                   ew1h_ref, tw1q_ref, ew1g_ref, tw1a_ref, eb1_ref,
                   emb_ref, p_ref, q_ref, c1_ref, t1g_ref, gacc):
    i = pl.program_id(0)
    agga = agga_ref[...]
    aggb = aggb_ref[...]
    agg = jnp.concatenate([agga[0], aggb[0], agga[1], aggb[1]], axis=1)
    deg = cnt_ref[...][:, 0:1]
    pre = x_ref[...] + agg / jnp.maximum(deg, 1.0)
    emb = jnp.tanh(
        jnp.dot(pre, w_ref[...], preferred_element_type=jnp.float32)
        + b_ref[...])
    emb_ref[...] = emb
    p_ref[...] = jnp.dot(emb, ew1h_ref[...],
                         preferred_element_type=jnp.float32)
    q_ref[...] = jnp.dot(emb, tw1q_ref[...],
                         preferred_element_type=jnp.float32)

    @pl.when(i == 0)
    def _():
        gacc[...] = jnp.zeros_like(gacc)

    gacc[...] += jnp.sum(emb, axis=0, keepdims=True)

    @pl.when(i == _NB - 1)
    def _():
        g = gacc[...] / jnp.float32(N)
        c1_ref[...] = jnp.dot(
            g, ew1g_ref[...], preferred_element_type=jnp.float32) + eb1_ref[...]
        t1g_ref[...] = jnp.dot(
            g, tw1a_ref[...], preferred_element_type=jnp.float32)


def _tc_dense(x, agga, aggb, cnt, w, b, ew1h, tw1q, ew1g, tw1a, eb1):
    return pl.pallas_call(
        _tc_dense_body,
        grid=(_NB,),
        in_specs=[
            pl.BlockSpec((_RB, D), lambda i: (i, 0)),
            pl.BlockSpec((NC, _RB, WQ), lambda i: (0, i, 0)),
            pl.BlockSpec((NC, _RB, WQ), lambda i: (0, i, 0)),
            pl.BlockSpec((_RB, 16), lambda i: (i, 0)),
            pl.BlockSpec((D, D), lambda i: (0, 0)),
            pl.BlockSpec((1, D), lambda i: (0, 0)),
            pl.BlockSpec((D, H), lambda i: (0, 0)),
            pl.BlockSpec((D, H), lambda i: (0, 0)),
            pl.BlockSpec((D, H), lambda i: (0, 0)),
            pl.BlockSpec((D, H), lambda i: (0, 0)),
            pl.BlockSpec((1, H), lambda i: (0, 0)),
        ],
        out_specs=[
            pl.BlockSpec((_RB, D), lambda i: (i, 0)),
            pl.BlockSpec((_RB, H), lambda i: (i, 0)),
            pl.BlockSpec((_RB, H), lambda i: (i, 0)),
            pl.BlockSpec((1, H), lambda i: (0, 0)),
            pl.BlockSpec((1, H), lambda i: (0, 0)),
        ],
        out_shape=[
            jax.ShapeDtypeStruct((N, D), jnp.float32),
            jax.ShapeDtypeStruct((N, H), jnp.float32),
            jax.ShapeDtypeStruct((N, H), jnp.float32),
            jax.ShapeDtypeStruct((1, H), jnp.float32),
            jax.ShapeDtypeStruct((1, H), jnp.float32),
        ],
        scratch_shapes=[pltpu.VMEM((1, D), jnp.float32)],
    )(x, agga, aggb, cnt, w, b, ew1h, tw1q, ew1g, tw1a, eb1)


@functools.cache
def _build_sc_pairs():
    @functools.partial(
        pl.kernel,
        out_type=(
            jax.ShapeDtypeStruct((EPAD, H), jnp.float32),
            jax.ShapeDtypeStruct((EPAD, H), jnp.float32),
        ),
        mesh=_sc_mesh(),
        compiler_params=pltpu.CompilerParams(
            use_tc_tiling_on_sc=False, needs_layout_passes=False),
        scratch_types=[
            pltpu.VMEM((NCH, CHUNK), jnp.int32),
            pltpu.VMEM((NCH, CHUNK), jnp.int32),
            pltpu.VMEM((EPT, H), jnp.float32),
            pltpu.VMEM_SHARED((NEXT, H), jnp.float32),
            pltpu.SemaphoreType.DMA,
        ],
    )
    def sc_pairs(p_hbm, src_hbm, dst_hbm, zs_out, zd_out, sidx, didx, zbuf,
                 p_sh, sem):
        c = lax.axis_index("c")
        s = lax.axis_index("s")
        wid = s * NC + c
        prt = NEXT // NS
        pltpu.sync_copy(p_hbm.at[pl.ds(s * prt, prt)],
                        p_sh.at[pl.ds(s * prt, prt)])
        pltpu.sync_copy(src_hbm.at[pl.ds(wid * NCH, NCH)], sidx)
        pltpu.sync_copy(dst_hbm.at[pl.ds(wid * NCH, NCH)], didx)
        plsc.subcore_barrier()

        def gather(idx, out_hbm):
            def fire(j, carry):
                pltpu.async_copy(p_sh.at[idx.at[j]],
                                 zbuf.at[pl.ds(j * CHUNK, CHUNK)], sem)
                return carry
            lax.fori_loop(0, NCH, fire, 0)

            def drain(j, carry):
                pltpu.make_async_copy(p_hbm.at[pl.ds(0, CHUNK)],
                                      zbuf.at[pl.ds(0, CHUNK)], sem).wait()
                return carry
            lax.fori_loop(0, NCH, drain, 0)
            pltpu.sync_copy(zbuf, out_hbm.at[pl.ds(wid * EPT, EPT)])

        gather(sidx, zs_out)
        gather(didx, zd_out)

    return sc_pairs


_ERB = 2048
_ENB = (EPAD // 8) // _ERB
_EROWS = E // 8


def _tc_edge_body(zs_ref, zd_ref, g1_ref, c1t_ref, w2t_ref, g8_ref,
                  srcr_ref, dstr_ref, emb_ref, tw1b_ref, tw1c_ref,
                  tb1_ref, t1g_ref,
                  vfs_ref, lpe_ref, c2_ref, smf, smi):
    i = pl.program_id(0)

    @pl.when(i == 0)
    def _():
        smf[0] = jnp.float32(-1e30)
        smf[1] = jnp.float32(0.0)
        smf[2] = jnp.float32(-3e38)
        smf[3] = jnp.float32(0.0)
        smi[0] = jnp.int32(0)

    z = zs_ref[...] + zd_ref[...] + c1t_ref[...]
    sig = 1.0 / (1.0 + jnp.exp(-z))
    scores = jnp.dot(sig * w2t_ref[...], g8_ref[...],
                     preferred_element_type=jnp.float32)

    rid = lax.broadcasted_iota(jnp.int32, (_ERB, 8), 0) + i * _ERB
    valid = rid < _EROWS
    sc = jnp.where(valid, scores, -1e30)

    bm = jnp.max(sc)
    m0 = smf[0]
    mn = jnp.maximum(m0, bm)
    seb = jnp.sum(jnp.exp(sc - mn))
    smf[1] = smf[1] * jnp.exp(m0 - mn) + seb
    smf[0] = mn

    eid = rid * 8 + lax.broadcasted_iota(jnp.int32, (_ERB, 8), 1)
    vv = jnp.where(valid, g1_ref[...] + scores, -3e38)
    bv = jnp.max(vv)
    bid = jnp.min(jnp.where(vv == bv, eid, jnp.int32(2**30)))
    bsc = jnp.max(jnp.where((vv == bv) & (eid == bid), scores, -3e38))

    @pl.when(bv > smf[2])
    def _():
        smf[2] = bv
        smf[3] = bsc
        smi[0] = bid

    @pl.when(i == _ENB - 1)
    def _():
        lse = smf[0] + jnp.log(smf[1])
        lpe_ref[...] = jnp.broadcast_to(smf[3] - lse, (1, 1))
        e_idx = smi[0]
        row = e_idx // 128
        col = e_idx % 128
        lanes = lax.broadcasted_iota(jnp.int32, (1, 128), 1)
        drow = dstr_ref[pl.ds(row, 1), :]
        srow = srcr_ref[pl.ds(row, 1), :]
        v_fir = jnp.sum(jnp.where(lanes == col, drow, 0))
        v_sec = jnp.sum(jnp.where(lanes == col, srow, 0))
        lanes2 = lax.broadcasted_iota(jnp.int32, (1, 2), 1)
        vfs_ref[...] = jnp.where(lanes2 == 0, v_fir, v_sec)
        evf = emb_ref[pl.ds(v_fir, 1), :]
        evs = emb_ref[pl.ds(v_sec, 1), :]
        c2_ref[...] = (
            t1g_ref[...]
            + jnp.dot(evf + evs, tw1b_ref[...],
                      preferred_element_type=jnp.float32)
            + jnp.dot(evf, tw1c_ref[...],
                      preferred_element_type=jnp.float32)
            + tb1_ref[...])


def _tc_edge(zs, zd, g1p, c1t, w2t, g8, srcr, dstr, emb, tw1b, tw1c, tb1,
             t1g):
    def full(shape):
        return pl.BlockSpec(shape, lambda i: tuple(0 for _ in shape))
    return pl.pallas_call(
        _tc_edge_body,
        grid=(_ENB,),
        in_specs=[
            pl.BlockSpec((_ERB, D), lambda i: (i, 0)),
            pl.BlockSpec((_ERB, D), lambda i: (i, 0)),
            pl.BlockSpec((_ERB, 8), lambda i: (i, 0)),
            full((1, D)),
            full((1, D)),
            full((D, 8)),
            full((E // 128, 128)),
            full((E // 128, 128)),
            full((N, D)),
            full((D, H)),
            full((D, H)),
            full((1, H)),
            full((1, H)),
        ],
        out_specs=[
            full((1, 2)),
            full((1, 1)),
            full((1, H)),
        ],
        out_shape=[
            jax.ShapeDtypeStruct((1, 2), jnp.int32),
            jax.ShapeDtypeStruct((1, 1), jnp.float32),
            jax.ShapeDtypeStruct((1, H), jnp.float32),
        ],
        scratch_shapes=[
            pltpu.SMEM((4,), jnp.float32),
            pltpu.SMEM((2,), jnp.int32),
        ],
    )(zs, zd, g1p, c1t, w2t, g8, srcr, dstr, emb, tw1b, tw1c, tb1, t1g)


@functools.cache
def _build_sc_mask():
    @functools.partial(
        pl.kernel,
        out_type=jax.ShapeDtypeStruct((NC, MROWS, 16), jnp.float32),
        mesh=_sc_mesh(),
        compiler_params=pltpu.CompilerParams(
            use_tc_tiling_on_sc=False, needs_layout_passes=False),
        scratch_types=[
            pltpu.VMEM((NCH, CHUNK), jnp.int32),
            pltpu.VMEM((NCH, CHUNK), jnp.int32),
            pltpu.VMEM((16,), jnp.int32),
            pltpu.VMEM((MROWS // CHUNK, CHUNK), jnp.int32),
            pltpu.VMEM((MROWS, 16), jnp.float32),
            pltpu.VMEM_SHARED((MROWS, 16), jnp.float32),
        ],
    )
    def sc_mask(src_hbm, dst_hbm, vf_hbm, idx_hbm, mask_out,
                sbuf, dbuf, vfb, idxb, mbuf, msh):
        c = lax.axis_index("c")
        s = lax.axis_index("s")
        wid = s * NC + c
        rows_per_tile = MROWS // NS

        def zbody(i, carry):
            mbuf[i, :] = jnp.zeros((16,), jnp.float32)
            return carry
        lax.fori_loop(0, MROWS, zbody, 0)

        pltpu.sync_copy(mbuf.at[pl.ds(s * rows_per_tile, rows_per_tile)],
                        msh.at[pl.ds(s * rows_per_tile, rows_per_tile)])
        pltpu.sync_copy(src_hbm.at[pl.ds(wid * NCH, NCH)], sbuf)
        pltpu.sync_copy(dst_hbm.at[pl.ds(wid * NCH, NCH)], dbuf)
        pltpu.sync_copy(vf_hbm, vfb)
        pltpu.sync_copy(idx_hbm, idxb)
        plsc.subcore_barrier()

        vfv = vfb[...]
        ones16 = jnp.ones((16,), jnp.float32)

        def body(t, carry):
            j = t // 8
            k = t % 8
            sv = sbuf[j, pl.ds(k * 16, 16)]
            dv = dbuf[j, pl.ds(k * 16, 16)]
            hit = sv == vfv
            plsc.store_scatter(mbuf, [dv // 16, dv % 16], ones16, mask=hit)
            return carry
        lax.fori_loop(0, NCH * 8, body, 0)
        plsc.subcore_barrier()

        def abody(k, carry):
            pltpu.sync_copy(mbuf.at[pl.ds(k * CHUNK, CHUNK)],
                            msh.at[idxb.at[k]], add=True)
            return carry
        lax.fori_loop(0, MROWS // CHUNK, abody, 0)
        plsc.subcore_barrier()

        pltpu.sync_copy(
            msh.at[pl.ds(s * rows_per_tile, rows_per_tile)],
            mask_out.at[c, pl.ds(s * rows_per_tile, rows_per_tile), :])

    return sc_mask


def _tc_third_body(q_ref, mask_ref, g2_ref, c2t_ref, w3t_ref, g8_ref,
                   vf_ref, lpe_ref, vthi_ref, lp_ref):
    z = q_ref[...] + c2t_ref[...]
    sig = 1.0 / (1.0 + jnp.exp(-z))
    scores = jnp.dot(sig * w3t_ref[...], g8_ref[...],
                     preferred_element_type=jnp.float32)
    hits = mask_ref[0] + mask_ref[1]
    nid = (lax.broadcasted_iota(jnp.int32, (N // 8, 8), 0) * 8
           + lax.broadcasted_iota(jnp.int32, (N // 8, 8), 1))
    vf = vf_ref[0, 0]
    masked = jnp.where((hits > 0.5) | (nid == vf), jnp.float32(-1e9), scores)
    m3 = jnp.max(masked)
    sh = masked - m3
    logp = sh - jnp.log(jnp.sum(jnp.exp(sh)))
    v = g2_ref[...] + logp
    bv = jnp.max(v)
    vthi = jnp.min(jnp.where(v == bv, nid, jnp.int32(2**30)))
    lp3 = jnp.max(jnp.where((v == bv) & (nid == vthi), logp,
                            jnp.float32(-3e38)))
    vthi_ref[...] = jnp.broadcast_to(vthi, (1, 1))
    lp_ref[...] = jnp.broadcast_to(lpe_ref[0, 0] + lp3, (1, 1))


def _tc_third(qr, mask2, g2r, c2t, w3t, g8, vfr, lpe):
    def full(shape):
        return pl.BlockSpec(shape, lambda: tuple(0 for _ in shape))
    return pl.pallas_call(
        _tc_third_body,
        in_specs=[
            full((N // 8, 128)),
            full((NC, N // 8, 8)),
            full((N // 8, 8)),
            full((1, 128)),
            full((1, 128)),
            full((128, 8)),
            full((1, 1)),
            full((1, 1)),
        ],
        out_specs=[full((1, 1)), full((1, 1))],
        out_shape=[
            jax.ShapeDtypeStruct((1, 1), jnp.int32),
            jax.ShapeDtypeStruct((1, 1), jnp.float32),
        ],
    )(qr, mask2, g2r, c2t, w3t, g8, vfr, lpe)


def kernel(x, edge_index, W_gnn, b_gnn, eW1, eb1, eW2, eb2, tW1, tb1, tW2,
           tb2):
    f32 = jnp.float32
    src = edge_index[0]
    dst = edge_index[1]
    pad = EPAD - E
    padv = jnp.full((pad,), N, jnp.int32)
    srcp = jnp.concatenate([src, padv]).reshape(EPAD // CHUNK, CHUNK)
    dstp = jnp.concatenate([dst, padv]).reshape(EPAD // CHUNK, CHUNK)
    rpad = jnp.zeros((NC, NEXT2 - N, WQ), f32)
    x4a = jnp.concatenate(
        [jnp.stack([x[:, 0 * WQ:1 * WQ], x[:, 2 * WQ:3 * WQ]]), rpad],
        axis=1)
    x4b = jnp.concatenate(
        [jnp.stack([x[:, 1 * WQ:2 * WQ], x[:, 3 * WQ:4 * WQ]]), rpad],
        axis=1)
    ids = jnp.arange(NEXT2, dtype=jnp.int32).reshape(NEXT2 // CHUNK, CHUNK)
    z32 = jnp.zeros((CHUNK, WQ), f32)
    z16 = jnp.zeros((CHUNK, 16), f32)
    agg_fn = _build_sc_aggregate()
    agga, cnta = agg_fn(x4a, srcp, dstp, ids, z32, z16)
    aggb, _ = agg_fn(x4b, srcp, dstp, ids, z32, z16)

    emb, P, Q, c1, t1g = _tc_dense(
        x, agga, aggb, cnta, W_gnn, b_gnn.reshape(1, D),
        eW1[D:2 * D], tW1[3 * D:4 * D], eW1[:D], tW1[:D],
        eb1.reshape(1, H))

    P_ext = jnp.concatenate([P, jnp.zeros((NEXT - N, H), f32)])
    Zs, Zd = _build_sc_pairs()(P_ext, srcp, dstp)

    g1 = jax.random.gumbel(jax.random.key(42), (E,), f32)
    g1p = jnp.concatenate([g1, jnp.full((pad,), -1e30, f32)])
    g1p = g1p.reshape(EPAD // 8, 8)
    g8 = (lax.broadcasted_iota(jnp.int32, (D, 8), 0) // H
          == lax.broadcasted_iota(jnp.int32, (D, 8), 1)).astype(f32)
    c1t = jnp.tile(c1, (1, 8))
    w2t = jnp.tile(eW2[:, 0].reshape(1, H), (1, 8))

    vfs, lpe, c2 = _tc_edge(
        Zs.reshape(EPAD // 8, D), Zd.reshape(EPAD // 8, D), g1p, c1t, w2t,
        g8, src.reshape(E // 128, 128), dst.reshape(E // 128, 128), emb,
        tW1[D:2 * D], tW1[2 * D:3 * D], tb1.reshape(1, H), t1g)

    vf16 = jnp.broadcast_to(vfs[0, 0], (16,)).astype(jnp.int32)
    idx640 = jnp.arange(MROWS, dtype=jnp.int32).reshape(MROWS // CHUNK, CHUNK)
    maskp = _build_sc_mask()(srcp, dstp, vf16, idx640)
    mask2 = maskp.reshape(NC, MROWS * 16)[:, :N].reshape(NC, N // 8, 8)

    g2 = jax.random.gumbel(jax.random.key(43), (N,), f32).reshape(N // 8, 8)
    c2t = jnp.tile(c2, (1, 8))
    w3t = jnp.tile(tW2[:, 0].reshape(1, H), (1, 8))

    vthi, lp = _tc_third(Q.reshape(N // 8, 128), mask2, g2, c2t, w3t, g8,
                         vfs[:, :1], lpe)

    action = jnp.stack([vfs[0, 0], vfs[0, 1], vthi[0, 0]])
    return action, lp[0, 0]


# speedup vs baseline: 1.1785x; 1.0554x over previous
import functools

import jax
import jax.numpy as jnp
from jax import lax
from jax.experimental import pallas as pl
from jax.experimental.pallas import tpu as pltpu
from jax.experimental.pallas import tpu_sc as plsc

N = 10000
D = 128
E = 160000
H = 16

NC = 2
NS = 16
NW = NC * NS
EPT = 5120
EPAD = EPT * NW
CHUNK = 128
NCH = EPT // CHUNK

NEXT = N + 16
ZROWS = 626
ACC_ROWS = NS * ZROWS
OUT_ROWS = N // NS
MROWS = 640


@functools.cache
def _sc_mesh():
    return plsc.VectorSubcoreMesh(core_axis_name="c", subcore_axis_name="s",
                                  num_cores=NC, num_subcores=NS)


DH = D // NC
WCH = DH + 16
TCH = EPAD // NS // CHUNK
NBUF = 4


WQ = D // NC // 2
NEXT2 = 10240
TSTG = NEXT2 // CHUNK // NS


@functools.cache
def _build_sc_aggregate(with_cnt=True):
    out_t = jax.ShapeDtypeStruct((NC, N, WQ), jnp.float32)
    if with_cnt:
        out_t = (out_t, jax.ShapeDtypeStruct((N, 16), jnp.float32))
    @functools.partial(
        pl.kernel,
        out_type=out_t,
        mesh=_sc_mesh(),
        compiler_params=pltpu.CompilerParams(
            use_tc_tiling_on_sc=False, needs_layout_passes=False),
        scratch_types=[
            pltpu.VMEM((TCH, CHUNK), jnp.int32),
            pltpu.VMEM((TCH, CHUNK), jnp.int32),
            [pltpu.VMEM((CHUNK, WQ), jnp.float32) for _ in range(NBUF)],
            pltpu.VMEM((TSTG, CHUNK), jnp.int32),
            pltpu.VMEM((CHUNK, WQ), jnp.float32),
            pltpu.VMEM((CHUNK, WQ), jnp.float32),
            pltpu.VMEM((CHUNK, 16), jnp.float32),
            pltpu.VMEM((CHUNK, 16), jnp.float32),
            pltpu.VMEM_SHARED((NEXT2, WQ), jnp.float32),
            pltpu.VMEM_SHARED((ACC_ROWS, WQ), jnp.float32),
            pltpu.VMEM_SHARED((ACC_ROWS, 16), jnp.float32),
            pltpu.SemaphoreType.DMA,
            pltpu.SemaphoreType.DMA,
            pltpu.SemaphoreType.DMA,
        ],
    )
    def sc_aggregate(x4_hbm, src_hbm, dst_hbm, ids_hbm, z_hbm, z16_hbm,
                     *rest):
        if with_cnt:
            (agg_out, cnt_out, sidx, didx, rows, idb, tbuf, zb, zb16,
             ones_v, tb_sh, acc_sh, cnt_sh, gsem, ssem, csem) = rest
        else:
            (agg_out, sidx, didx, rows, idb, tbuf, zb, zb16,
             ones_v, tb_sh, acc_sh, cnt_sh, gsem, ssem, csem) = rest
        c = lax.axis_index("c")
        s = lax.axis_index("s")

        pltpu.sync_copy(ids_hbm.at[pl.ds(s * TSTG, TSTG)], idb)

        def stage(k, carry):
            j = s * TSTG + k
            pltpu.async_copy(x4_hbm.at[c].at[idb.at[k]], tbuf, gsem).wait()
            pltpu.sync_copy(tbuf, tb_sh.at[pl.ds(j * CHUNK, CHUNK)])
            return carry
        lax.fori_loop(0, TSTG, stage, 0)

        if with_cnt:
            def fill_ones(i, carry):
                ones_v[i, :] = jnp.ones((16,), jnp.float32)
                return carry
            lax.fori_loop(0, CHUNK, fill_ones, 0)

        pltpu.sync_copy(z_hbm, zb)
        tail = ZROWS - 4 * CHUNK
        for k in range(4):
            pltpu.sync_copy(zb, acc_sh.at[pl.ds(s * ZROWS + k * CHUNK,
                                                CHUNK)])
        pltpu.sync_copy(zb.at[pl.ds(0, tail)],
                        acc_sh.at[pl.ds(s * ZROWS + 4 * CHUNK, tail)])
        if with_cnt:
            pltpu.sync_copy(z16_hbm, zb16)
            for k in range(4):
                pltpu.sync_copy(zb16, cnt_sh.at[pl.ds(s * ZROWS + k * CHUNK,
                                                      CHUNK)])
            pltpu.sync_copy(zb16.at[pl.ds(0, tail)],
                            cnt_sh.at[pl.ds(s * ZROWS + 4 * CHUNK, tail)])
        pltpu.sync_copy(src_hbm.at[pl.ds(s * TCH, TCH)], sidx)
        pltpu.sync_copy(dst_hbm.at[pl.ds(s * TCH, TCH)], didx)
        plsc.subcore_barrier()

        def gather(j, b):
            pltpu.async_copy(tb_sh.at[sidx.at[j]], rows[b], gsem)

        def scatter(j, b):
            pltpu.async_copy(rows[b], acc_sh.at[didx.at[j]], ssem, add=True)
            if with_cnt:
                pltpu.async_copy(ones_v, cnt_sh.at[didx.at[j]], csem,
                                 add=True)

        def wait_gather(b):
            pltpu.make_async_copy(z_hbm, rows[b], gsem).wait()

        def wait_scatter(b):
            pltpu.make_async_copy(z_hbm, rows[b], ssem).wait()

        for b in range(NBUF):
            gather(b, b)

        def group(g, carry):
            base = g * NBUF
            for b in range(NBUF):
                wait_gather(b)
                scatter(base + b, b)
            for b in range(NBUF):
                wait_scatter(b)

                @pl.when(base + b + NBUF < TCH)
                def _(b=b):
                    gather(base + b + NBUF, b)

            return carry

        lax.fori_loop(0, TCH // NBUF, group, 0)

        if with_cnt:
            def cdrain(j, carry):
                pltpu.make_async_copy(z16_hbm, ones_v, csem).wait()
                return carry
            lax.fori_loop(0, TCH, cdrain, 0)
        plsc.subcore_barrier()

        pltpu.sync_copy(acc_sh.at[pl.ds(s * OUT_ROWS, OUT_ROWS)],
                        agg_out.at[c, pl.ds(s * OUT_ROWS, OUT_ROWS), :])

        if with_cnt:
            @pl.when(c == 0)
            def _():
                pltpu.sync_copy(cnt_sh.at[pl.ds(s * OUT_ROWS, OUT_ROWS)],
                                cnt_out.at[pl.ds(s * OUT_ROWS, OUT_ROWS), :])

    return sc_aggregate


_RB = 400
_NB = N // _RB


def _tc_dense_body(x_ref, agga_ref, aggb_ref, cnt_ref, w_ref, b_ref,
                   ew1h_ref, tw1q_ref, ew1g_ref, tw1a_ref, eb1_ref,
                   emb_ref, p_ref, q_ref, c1_ref, t1g_ref, gacc):
    i = pl.program_id(0)
    agga = agga_ref[...]
    aggb = aggb_ref[...]
    agg = jnp.concatenate([agga[0], aggb[0], agga[1], aggb[1]], axis=1)
    deg = cnt_ref[...][:, 0:1]
    pre = x_ref[...] + agg / jnp.maximum(deg, 1.0)
    emb = jnp.tanh(
        jnp.dot(pre, w_ref[...], preferred_element_type=jnp.float32)
        + b_ref[...])
    emb_ref[...] = emb
    p_ref[...] = jnp.dot(emb, ew1h_ref[...],
                         preferred_element_type=jnp.float32)
    q_ref[...] = jnp.dot(emb, tw1q_ref[...],
                         preferred_element_type=jnp.float32)

    @pl.when(i == 0)
    def _():
        gacc[...] = jnp.zeros_like(gacc)

    gacc[...] += jnp.sum(emb, axis=0, keepdims=True)

    @pl.when(i == _NB - 1)
    def _():
        g = gacc[...] / jnp.float32(N)
        c1_ref[...] = jnp.dot(
            g, ew1g_ref[...], preferred_element_type=jnp.float32) + eb1_ref[...]
        t1g_ref[...] = jnp.dot(
            g, tw1a_ref[...], preferred_element_type=jnp.float32)


def _tc_dense(x, agga, aggb, cnt, w, b, ew1h, tw1q, ew1g, tw1a, eb1):
    return pl.pallas_call(
        _tc_dense_body,
        grid=(_NB,),
        in_specs=[
            pl.BlockSpec((_RB, D), lambda i: (i, 0)),
            pl.BlockSpec((NC, _RB, WQ), lambda i: (0, i, 0)),
            pl.BlockSpec((NC, _RB, WQ), lambda i: (0, i, 0)),
            pl.BlockSpec((_RB, 16), lambda i: (i, 0)),
            pl.BlockSpec((D, D), lambda i: (0, 0)),
            pl.BlockSpec((1, D), lambda i: (0, 0)),
            pl.BlockSpec((D, H), lambda i: (0, 0)),
            pl.BlockSpec((D, H), lambda i: (0, 0)),
            pl.BlockSpec((D, H), lambda i: (0, 0)),
            pl.BlockSpec((D, H), lambda i: (0, 0)),
            pl.BlockSpec((1, H), lambda i: (0, 0)),
        ],
        out_specs=[
            pl.BlockSpec((_RB, D), lambda i: (i, 0)),
            pl.BlockSpec((_RB, H), lambda i: (i, 0)),
            pl.BlockSpec((_RB, H), lambda i: (i, 0)),
            pl.BlockSpec((1, H), lambda i: (0, 0)),
            pl.BlockSpec((1, H), lambda i: (0, 0)),
        ],
        out_shape=[
            jax.ShapeDtypeStruct((N, D), jnp.float32),
            jax.ShapeDtypeStruct((N, H), jnp.float32),
            jax.ShapeDtypeStruct((N, H), jnp.float32),
            jax.ShapeDtypeStruct((1, H), jnp.float32),
            jax.ShapeDtypeStruct((1, H), jnp.float32),
        ],
        scratch_shapes=[pltpu.VMEM((1, D), jnp.float32)],
    )(x, agga, aggb, cnt, w, b, ew1h, tw1q, ew1g, tw1a, eb1)


@functools.cache
def _build_sc_pairs():
    @functools.partial(
        pl.kernel,
        out_type=(
            jax.ShapeDtypeStruct((EPAD, H), jnp.float32),
            jax.ShapeDtypeStruct((EPAD, H), jnp.float32),
        ),
        mesh=_sc_mesh(),
        compiler_params=pltpu.CompilerParams(
            use_tc_tiling_on_sc=False, needs_layout_passes=False),
        scratch_types=[
            pltpu.VMEM((NCH, CHUNK), jnp.int32),
            pltpu.VMEM((NCH, CHUNK), jnp.int32),
            pltpu.VMEM((EPT, H), jnp.float32),
            pltpu.VMEM_SHARED((NEXT, H), jnp.float32),
            pltpu.SemaphoreType.DMA,
        ],
    )
    def sc_pairs(p_hbm, src_hbm, dst_hbm, zs_out, zd_out, sidx, didx, zbuf,
                 p_sh, sem):
        c = lax.axis_index("c")
        s = lax.axis_index("s")
        wid = s * NC + c
        prt = NEXT // NS
        pltpu.sync_copy(p_hbm.at[pl.ds(s * prt, prt)],
                        p_sh.at[pl.ds(s * prt, prt)])
        pltpu.sync_copy(src_hbm.at[pl.ds(wid * NCH, NCH)], sidx)
        pltpu.sync_copy(dst_hbm.at[pl.ds(wid * NCH, NCH)], didx)
        plsc.subcore_barrier()

        def gather(idx, out_hbm):
            def fire(j, carry):
                pltpu.async_copy(p_sh.at[idx.at[j]],
                                 zbuf.at[pl.ds(j * CHUNK, CHUNK)], sem)
                return carry
            lax.fori_loop(0, NCH, fire, 0)

            def drain(j, carry):
                pltpu.make_async_copy(p_hbm.at[pl.ds(0, CHUNK)],
                                      zbuf.at[pl.ds(0, CHUNK)], sem).wait()
                return carry
            lax.fori_loop(0, NCH, drain, 0)
            pltpu.sync_copy(zbuf, out_hbm.at[pl.ds(wid * EPT, EPT)])

        gather(sidx, zs_out)
        gather(didx, zd_out)

    return sc_pairs


_ERB = 2048
_ENB = (EPAD // 8) // _ERB
_EROWS = E // 8


def _tc_edge_body(zs_ref, zd_ref, g1_ref, c1t_ref, w2t_ref, g8_ref,
                  srcr_ref, dstr_ref, emb_ref, tw1b_ref, tw1c_ref,
                  tb1_ref, t1g_ref,
                  vfs_ref, lpe_ref, c2_ref, smf, smi):
    i = pl.program_id(0)

    @pl.when(i == 0)
    def _():
        smf[0] = jnp.float32(-1e30)
        smf[1] = jnp.float32(0.0)
        smf[2] = jnp.float32(-3e38)
        smf[3] = jnp.float32(0.0)
        smi[0] = jnp.int32(0)

    z = zs_ref[...] + zd_ref[...] + c1t_ref[...]
    sig = 1.0 / (1.0 + jnp.exp(-z))
    scores = jnp.dot(sig * w2t_ref[...], g8_ref[...],
                     preferred_element_type=jnp.float32)

    rid = lax.broadcasted_iota(jnp.int32, (_ERB, 8), 0) + i * _ERB
    valid = rid < _EROWS
    sc = jnp.where(valid, scores, -1e30)

    bm = jnp.max(sc)
    m0 = smf[0]
    mn = jnp.maximum(m0, bm)
    seb = jnp.sum(jnp.exp(sc - mn))
    smf[1] = smf[1] * jnp.exp(m0 - mn) + seb
    smf[0] = mn

    eid = rid * 8 + lax.broadcasted_iota(jnp.int32, (_ERB, 8), 1)
    vv = jnp.where(valid, g1_ref[...] + scores, -3e38)
    bv = jnp.max(vv)
    bid = jnp.min(jnp.where(vv == bv, eid, jnp.int32(2**30)))
    bsc = jnp.max(jnp.where((vv == bv) & (eid == bid), scores, -3e38))

    @pl.when(bv > smf[2])
    def _():
        smf[2] = bv
        smf[3] = bsc
        smi[0] = bid

    @pl.when(i == _ENB - 1)
    def _():
        lse = smf[0] + jnp.log(smf[1])
        lpe_ref[...] = jnp.broadcast_to(smf[3] - lse, (1, 1))
        e_idx = smi[0]
        row = e_idx // 128
        col = e_idx % 128
        lanes = lax.broadcasted_iota(jnp.int32, (1, 128), 1)
        drow = dstr_ref[pl.ds(row, 1), :]
        srow = srcr_ref[pl.ds(row, 1), :]
        v_fir = jnp.sum(jnp.where(lanes == col, drow, 0))
        v_sec = jnp.sum(jnp.where(lanes == col, srow, 0))
        lanes2 = lax.broadcasted_iota(jnp.int32, (1, 2), 1)
        vfs_ref[...] = jnp.where(lanes2 == 0, v_fir, v_sec)
        evf = emb_ref[pl.ds(v_fir, 1), :]
        evs = emb_ref[pl.ds(v_sec, 1), :]
        c2_ref[...] = (
            t1g_ref[...]
            + jnp.dot(evf + evs, tw1b_ref[...],
                      preferred_element_type=jnp.float32)
            + jnp.dot(evf, tw1c_ref[...],
                      preferred_element_type=jnp.float32)
            + tb1_ref[...])


def _tc_edge(zs, zd, g1p, c1t, w2t, g8, srcr, dstr, emb, tw1b, tw1c, tb1,
             t1g):
    def full(shape):
        return pl.BlockSpec(shape, lambda i: tuple(0 for _ in shape))
    return pl.pallas_call(
        _tc_edge_body,
        grid=(_ENB,),
        in_specs=[
            pl.BlockSpec((_ERB, D), lambda i: (i, 0)),
            pl.BlockSpec((_ERB, D), lambda i: (i, 0)),
            pl.BlockSpec((_ERB, 8), lambda i: (i, 0)),
            full((1, D)),
            full((1, D)),
            full((D, 8)),
            full((E // 128, 128)),
            full((E // 128, 128)),
            full((N, D)),
            full((D, H)),
            full((D, H)),
            full((1, H)),
            full((1, H)),
        ],
        out_specs=[
            full((1, 2)),
            full((1, 1)),
            full((1, H)),
        ],
        out_shape=[
            jax.ShapeDtypeStruct((1, 2), jnp.int32),
            jax.ShapeDtypeStruct((1, 1), jnp.float32),
            jax.ShapeDtypeStruct((1, H), jnp.float32),
        ],
        scratch_shapes=[
            pltpu.SMEM((4,), jnp.float32),
            pltpu.SMEM((2,), jnp.int32),
        ],
    )(zs, zd, g1p, c1t, w2t, g8, srcr, dstr, emb, tw1b, tw1c, tb1, t1g)


@functools.cache
def _build_sc_mask():
    @functools.partial(
        pl.kernel,
        out_type=jax.ShapeDtypeStruct((NC, MROWS, 16), jnp.float32),
        mesh=_sc_mesh(),
        compiler_params=pltpu.CompilerParams(
            use_tc_tiling_on_sc=False, needs_layout_passes=False),
        scratch_types=[
            pltpu.VMEM((NCH, CHUNK), jnp.int32),
            pltpu.VMEM((NCH, CHUNK), jnp.int32),
            pltpu.VMEM((16,), jnp.int32),
            pltpu.VMEM((MROWS // CHUNK, CHUNK), jnp.int32),
            pltpu.VMEM((MROWS, 16), jnp.float32),
            pltpu.VMEM_SHARED((MROWS, 16), jnp.float32),
        ],
    )
    def sc_mask(src_hbm, dst_hbm, vf_hbm, idx_hbm, mask_out,
                sbuf, dbuf, vfb, idxb, mbuf, msh):
        c = lax.axis_index("c")
        s = lax.axis_index("s")
        wid = s * NC + c
        rows_per_tile = MROWS // NS

        def zbody(i, carry):
            mbuf[i, :] = jnp.zeros((16,), jnp.float32)
            return carry
        lax.fori_loop(0, MROWS, zbody, 0)

        pltpu.sync_copy(mbuf.at[pl.ds(s * rows_per_tile, rows_per_tile)],
                        msh.at[pl.ds(s * rows_per_tile, rows_per_tile)])
        pltpu.sync_copy(src_hbm.at[pl.ds(wid * NCH, NCH)], sbuf)
        pltpu.sync_copy(dst_hbm.at[pl.ds(wid * NCH, NCH)], dbuf)
        pltpu.sync_copy(vf_hbm, vfb)
        pltpu.sync_copy(idx_hbm, idxb)
        plsc.subcore_barrier()

        vfv = vfb[...]
        ones16 = jnp.ones((16,), jnp.float32)

        def body(t, carry):
            j = t // 8
            k = t % 8
            sv = sbuf[j, pl.ds(k * 16, 16)]
            dv = dbuf[j, pl.ds(k * 16, 16)]
            hit = sv == vfv
            plsc.store_scatter(mbuf, [dv // 16, dv % 16], ones16, mask=hit)
            return carry
        lax.fori_loop(0, NCH * 8, body, 0)
        plsc.subcore_barrier()

        def abody(k, carry):
            pltpu.sync_copy(mbuf.at[pl.ds(k * CHUNK, CHUNK)],
                            msh.at[idxb.at[k]], add=True)
            return carry
        lax.fori_loop(0, MROWS // CHUNK, abody, 0)
        plsc.subcore_barrier()

        pltpu.sync_copy(
            msh.at[pl.ds(s * rows_per_tile, rows_per_tile)],
            mask_out.at[c, pl.ds(s * rows_per_tile, rows_per_tile), :])

    return sc_mask


def _tc_third_body(q_ref, mask_ref, g2_ref, c2t_ref, w3t_ref, g8_ref,
                   vf_ref, lpe_ref, vthi_ref, lp_ref):
    z = q_ref[...] + c2t_ref[...]
    sig = 1.0 / (1.0 + jnp.exp(-z))
    scores = jnp.dot(sig * w3t_ref[...], g8_ref[...],
                     preferred_element_type=jnp.float32)
    hits = mask_ref[0] + mask_ref[1]
    nid = (lax.broadcasted_iota(jnp.int32, (N // 8, 8), 0) * 8
           + lax.broadcasted_iota(jnp.int32, (N // 8, 8), 1))
    vf = vf_ref[0, 0]
    masked = jnp.where((hits > 0.5) | (nid == vf), jnp.float32(-1e9), scores)
    m3 = jnp.max(masked)
    sh = masked - m3
    logp = sh - jnp.log(jnp.sum(jnp.exp(sh)))
    v = g2_ref[...] + logp
    bv = jnp.max(v)
    vthi = jnp.min(jnp.where(v == bv, nid, jnp.int32(2**30)))
    lp3 = jnp.max(jnp.where((v == bv) & (nid == vthi), logp,
                            jnp.float32(-3e38)))
    vthi_ref[...] = jnp.broadcast_to(vthi, (1, 1))
    lp_ref[...] = jnp.broadcast_to(lpe_ref[0, 0] + lp3, (1, 1))


def _tc_third(qr, mask2, g2r, c2t, w3t, g8, vfr, lpe):
    def full(shape):
        return pl.BlockSpec(shape, lambda: tuple(0 for _ in shape))
    return pl.pallas_call(
        _tc_third_body,
        in_specs=[
            full((N // 8, 128)),
            full((NC, N // 8, 8)),
            full((N // 8, 8)),
            full((1, 128)),
            full((1, 128)),
            full((128, 8)),
            full((1, 1)),
            full((1, 1)),
        ],
        out_specs=[full((1, 1)), full((1, 1))],
        out_shape=[
            jax.ShapeDtypeStruct((1, 1), jnp.int32),
            jax.ShapeDtypeStruct((1, 1), jnp.float32),
        ],
    )(qr, mask2, g2r, c2t, w3t, g8, vfr, lpe)


def kernel(x, edge_index, W_gnn, b_gnn, eW1, eb1, eW2, eb2, tW1, tb1, tW2,
           tb2):
    f32 = jnp.float32
    src = edge_index[0]
    dst = edge_index[1]
    pad = EPAD - E
    padv = jnp.full((pad,), N, jnp.int32)
    srcp = jnp.concatenate([src, padv]).reshape(EPAD // CHUNK, CHUNK)
    dstp = jnp.concatenate([dst, padv]).reshape(EPAD // CHUNK, CHUNK)
    rpad = jnp.zeros((NC, NEXT2 - N, WQ), f32)
    x4a = jnp.concatenate(
        [jnp.stack([x[:, 0 * WQ:1 * WQ], x[:, 2 * WQ:3 * WQ]]), rpad],
        axis=1)
    x4b = jnp.concatenate(
        [jnp.stack([x[:, 1 * WQ:2 * WQ], x[:, 3 * WQ:4 * WQ]]), rpad],
        axis=1)
    ids = jnp.arange(NEXT2, dtype=jnp.int32).reshape(NEXT2 // CHUNK, CHUNK)
    z32 = jnp.zeros((CHUNK, WQ), f32)
    z16 = jnp.zeros((CHUNK, 16), f32)
    agga, cnta = _build_sc_aggregate(True)(x4a, srcp, dstp, ids, z32, z16)
    aggb = _build_sc_aggregate(False)(x4b, srcp, dstp, ids, z32, z16)

    emb, P, Q, c1, t1g = _tc_dense(
        x, agga, aggb, cnta, W_gnn, b_gnn.reshape(1, D),
        eW1[D:2 * D], tW1[3 * D:4 * D], eW1[:D], tW1[:D],
        eb1.reshape(1, H))

    P_ext = jnp.concatenate([P, jnp.zeros((NEXT - N, H), f32)])
    Zs, Zd = _build_sc_pairs()(P_ext, srcp, dstp)

    g1 = jax.random.gumbel(jax.random.key(42), (E,), f32)
    g1p = jnp.concatenate([g1, jnp.full((pad,), -1e30, f32)])
    g1p = g1p.reshape(EPAD // 8, 8)
    g8 = (lax.broadcasted_iota(jnp.int32, (D, 8), 0) // H
          == lax.broadcasted_iota(jnp.int32, (D, 8), 1)).astype(f32)
    c1t = jnp.tile(c1, (1, 8))
    w2t = jnp.tile(eW2[:, 0].reshape(1, H), (1, 8))

    vfs, lpe, c2 = _tc_edge(
        Zs.reshape(EPAD // 8, D), Zd.reshape(EPAD // 8, D), g1p, c1t, w2t,
        g8, src.reshape(E // 128, 128), dst.reshape(E // 128, 128), emb,
        tW1[D:2 * D], tW1[2 * D:3 * D], tb1.reshape(1, H), t1g)

    vf16 = jnp.broadcast_to(vfs[0, 0], (16,)).astype(jnp.int32)
    idx640 = jnp.arange(MROWS, dtype=jnp.int32).reshape(MROWS // CHUNK, CHUNK)
    maskp = _build_sc_mask()(srcp, dstp, vf16, idx640)
    mask2 = maskp.reshape(NC, MROWS * 16)[:, :N].reshape(NC, N // 8, 8)

    g2 = jax.random.gumbel(jax.random.key(43), (N,), f32).reshape(N // 8, 8)
    c2t = jnp.tile(c2, (1, 8))
    w3t = jnp.tile(tW2[:, 0].reshape(1, H), (1, 8))

    vthi, lp = _tc_third(Q.reshape(N // 8, 128), mask2, g2, c2t, w3t, g8,
                         vfs[:, :1], lpe)

    action = jnp.stack([vfs[0, 0], vfs[0, 1], vthi[0, 0]])
    return action, lp[0, 0]
